# Initial kernel scaffold; baseline (speedup 1.0000x reference)
import jax
import jax.numpy as jnp
from jax.experimental import pallas as pl


def kernel(x, edge_index, W1l, b1l, W1r, W2l, b2l, W2r, W3, b3, W4, b4):
    raise NotImplementedError("write your pallas kernel here")


# speedup vs baseline: 5.9057x; 5.9057x over previous
import functools

import jax
import jax.numpy as jnp
from jax import lax
from jax.experimental import pallas as pl
from jax.experimental.pallas import tpu as pltpu
from jax.experimental.pallas import tpu_sc as plsc

_NC = 2
_NS = 16
_NW = _NC * _NS
_CB = 128


def _cdiv(a, b):
  return (a + b - 1) // b


def _mm2(x, wl, wr, rb):
  m, k = x.shape
  dl, dr = wl.shape[1], wr.shape[1]

  def body(x_ref, wl_ref, wr_ref, ol_ref, or_ref):
    xv = x_ref[...]
    ol_ref[...] = jnp.dot(xv, wl_ref[...], preferred_element_type=jnp.float32)
    or_ref[...] = jnp.dot(xv, wr_ref[...], preferred_element_type=jnp.float32)

  return pl.pallas_call(
      body,
      grid=(m // rb,),
      in_specs=[
          pl.BlockSpec((rb, k), lambda i: (i, 0)),
          pl.BlockSpec((k, dl), lambda i: (0, 0)),
          pl.BlockSpec((k, dr), lambda i: (0, 0)),
      ],
      out_specs=[
          pl.BlockSpec((rb, dl), lambda i: (i, 0)),
          pl.BlockSpec((rb, dr), lambda i: (i, 0)),
      ],
      out_shape=[
          jax.ShapeDtypeStruct((m, dl), jnp.float32),
          jax.ShapeDtypeStruct((m, dr), jnp.float32),
      ],
  )(x, wl, wr)


def _combine_mm2(a0, a1, d0, d1, r, b, wl, wr, rb, relu):
  m, dh = r.shape
  dl, dr = wl.shape[1], wr.shape[1]

  def body(a0_ref, a1_ref, d0_ref, d1_ref, r_ref, b_ref, wl_ref, wr_ref,
           ol_ref, or_ref):
    deg = jnp.maximum((d0_ref[...] + d1_ref[...])[:, :1], 1.0)
    h = (a0_ref[...] + a1_ref[...]) / deg + r_ref[...] + b_ref[...]
    if relu:
      h = jnp.maximum(h, 0.0)
    ol_ref[...] = jnp.dot(h, wl_ref[...], preferred_element_type=jnp.float32)
    or_ref[...] = jnp.dot(h, wr_ref[...], preferred_element_type=jnp.float32)

  return pl.pallas_call(
      body,
      grid=(m // rb,),
      in_specs=[
          pl.BlockSpec((rb, dh), lambda i: (i, 0)),
          pl.BlockSpec((rb, dh), lambda i: (i, 0)),
          pl.BlockSpec((rb, 16), lambda i: (i, 0)),
          pl.BlockSpec((rb, 16), lambda i: (i, 0)),
          pl.BlockSpec((rb, dh), lambda i: (i, 0)),
          pl.BlockSpec((1, dh), lambda i: (0, 0)),
          pl.BlockSpec((dh, dl), lambda i: (0, 0)),
          pl.BlockSpec((dh, dr), lambda i: (0, 0)),
      ],
      out_specs=[
          pl.BlockSpec((rb, dl), lambda i: (i, 0)),
          pl.BlockSpec((rb, dr), lambda i: (i, 0)),
      ],
      out_shape=[
          jax.ShapeDtypeStruct((m, dl), jnp.float32),
          jax.ShapeDtypeStruct((m, dr), jnp.float32),
      ],
  )(a0, a1, d0, d1, r, b, wl, wr)


def _combine_z(a0, a1, d0, d1, r, b, rb):
  m, dh = r.shape

  def body(a0_ref, a1_ref, d0_ref, d1_ref, r_ref, b_ref, o_ref):
    deg = jnp.maximum((d0_ref[...] + d1_ref[...])[:, :1], 1.0)
    o_ref[...] = (a0_ref[...] + a1_ref[...]) / deg + r_ref[...] + b_ref[...]

  return pl.pallas_call(
      body,
      grid=(m // rb,),
      in_specs=[
          pl.BlockSpec((rb, dh), lambda i: (i, 0)),
          pl.BlockSpec((rb, dh), lambda i: (i, 0)),
          pl.BlockSpec((rb, 16), lambda i: (i, 0)),
          pl.BlockSpec((rb, 16), lambda i: (i, 0)),
          pl.BlockSpec((rb, dh), lambda i: (i, 0)),
          pl.BlockSpec((1, dh), lambda i: (0, 0)),
      ],
      out_specs=pl.BlockSpec((rb, dh), lambda i: (i, 0)),
      out_shape=jax.ShapeDtypeStruct((m, dh), jnp.float32),
  )(a0, a1, d0, d1, r, b)


def _decode(ca, cb, w3a, w3b, b3, w4, b4, rb):
  m = ca.shape[0]

  def body(ca_ref, cb_ref, w3a_ref, w3b_ref, b3_ref, w4_ref, b4_ref, o_ref):
    hid = jnp.maximum(
        jnp.dot(ca_ref[...], w3a_ref[...], preferred_element_type=jnp.float32)
        + jnp.dot(cb_ref[...], w3b_ref[...], preferred_element_type=jnp.float32)
        + b3_ref[...], 0.0)
    o_ref[...] = jnp.sum(hid * w4_ref[...], axis=1) + b4_ref[0, 0]

  return pl.pallas_call(
      body,
      grid=(m // rb,),
      in_specs=[
          pl.BlockSpec((rb, 16), lambda i: (i, 0)),
          pl.BlockSpec((rb, 16), lambda i: (i, 0)),
          pl.BlockSpec((16, 16), lambda i: (0, 0)),
          pl.BlockSpec((16, 16), lambda i: (0, 0)),
          pl.BlockSpec((1, 16), lambda i: (0, 0)),
          pl.BlockSpec((1, 16), lambda i: (0, 0)),
          pl.BlockSpec((1, 1), lambda i: (0, 0)),
      ],
      out_specs=pl.BlockSpec((rb,), lambda i: (i,)),
      out_shape=jax.ShapeDtypeStruct((m,), jnp.float32),
  )(ca, cb, w3a, w3b, b3, w4, b4)


def _zero_rows(buf, rows, d):
  z = jnp.zeros((16,), jnp.float32)

  def zf(i, c):
    for kk in range(d // 16):
      buf[i, pl.ds(kk * 16, 16)] = z
    return c

  lax.fori_loop(0, rows, zf, 0)


def _seg_sum_deg(p, src3, dst3, n_pad):
  d = p.shape[1]
  nw, nchunk, cb = src3.shape
  rpt = n_pad // _NS
  mesh = plsc.VectorSubcoreMesh(core_axis_name="c", subcore_axis_name="s", num_cores=_NC, num_subcores=_NS)

  @functools.partial(
      pl.kernel,
      out_type=[
          jax.ShapeDtypeStruct((_NC * n_pad, d), jnp.float32),
          jax.ShapeDtypeStruct((_NC * n_pad, 16), jnp.float32),
      ],
      mesh=mesh,
      compiler_params=pltpu.CompilerParams(use_tc_tiling_on_sc=False),
      scratch_types=[
          pltpu.VMEM_SHARED((n_pad, d), jnp.float32),
          pltpu.VMEM_SHARED((n_pad, 16), jnp.float32),
          pltpu.VMEM((nchunk, cb), jnp.int32),
          pltpu.VMEM((nchunk, cb), jnp.int32),
          pltpu.VMEM((cb, d), jnp.float32),
          pltpu.VMEM((cb, 16), jnp.float32),
          pltpu.SemaphoreType.DMA,
      ],
  )
  def k(p_hbm, src_hbm, dst_hbm, agg_out, deg_out,
        agg_sp, deg_sp, sidx, didx, gbuf, ones, sem):
    c = lax.axis_index("c")
    s = lax.axis_index("s")
    w = s * _NC + c

    _zero_rows(gbuf, cb, d)
    _zero_rows(ones, cb, 16)
    for kk in range(rpt // cb):
      pltpu.sync_copy(gbuf, agg_sp.at[pl.ds(s * rpt + kk * cb, cb)])
      pltpu.sync_copy(ones, deg_sp.at[pl.ds(s * rpt + kk * cb, cb)])

    one = jnp.ones((16,), jnp.float32)

    def of(i, cacc):
      ones[i, pl.ds(0, 16)] = one
      return cacc

    lax.fori_loop(0, cb, of, 0)

    pltpu.sync_copy(src_hbm.at[w], sidx)
    pltpu.sync_copy(dst_hbm.at[w], didx)
    plsc.subcore_barrier()

    def body(j, cacc):
      pltpu.async_copy(p_hbm.at[sidx.at[j]], gbuf, sem).wait()
      pltpu.sync_copy(gbuf, agg_sp.at[didx.at[j]], add=True)
      pltpu.sync_copy(ones, deg_sp.at[didx.at[j]], add=True)
      return cacc

    lax.fori_loop(0, nchunk, body, 0)
    plsc.subcore_barrier()

    pltpu.sync_copy(agg_sp.at[pl.ds(s * rpt, rpt)],
                    agg_out.at[pl.ds(c * n_pad + s * rpt, rpt)])
    pltpu.sync_copy(deg_sp.at[pl.ds(s * rpt, rpt)],
                    deg_out.at[pl.ds(c * n_pad + s * rpt, rpt)])

  return k(p, src3, dst3)


def _seg_sum(p, src3, dst3, n_pad):
  d = p.shape[1]
  nw, nchunk, cb = src3.shape
  rpt = n_pad // _NS
  mesh = plsc.VectorSubcoreMesh(core_axis_name="c", subcore_axis_name="s", num_cores=_NC, num_subcores=_NS)

  @functools.partial(
      pl.kernel,
      out_type=jax.ShapeDtypeStruct((_NC * n_pad, d), jnp.float32),
      mesh=mesh,
      compiler_params=pltpu.CompilerParams(use_tc_tiling_on_sc=False),
      scratch_types=[
          pltpu.VMEM_SHARED((n_pad, d), jnp.float32),
          pltpu.VMEM((nchunk, cb), jnp.int32),
          pltpu.VMEM((nchunk, cb), jnp.int32),
          pltpu.VMEM((cb, d), jnp.float32),
          pltpu.SemaphoreType.DMA,
      ],
  )
  def k(p_hbm, src_hbm, dst_hbm, agg_out, agg_sp, sidx, didx, gbuf, sem):
    c = lax.axis_index("c")
    s = lax.axis_index("s")
    w = s * _NC + c

    _zero_rows(gbuf, cb, d)
    for kk in range(rpt // cb):
      pltpu.sync_copy(gbuf, agg_sp.at[pl.ds(s * rpt + kk * cb, cb)])

    pltpu.sync_copy(src_hbm.at[w], sidx)
    pltpu.sync_copy(dst_hbm.at[w], didx)
    plsc.subcore_barrier()

    def body(j, cacc):
      pltpu.async_copy(p_hbm.at[sidx.at[j]], gbuf, sem).wait()
      pltpu.sync_copy(gbuf, agg_sp.at[didx.at[j]], add=True)
      return cacc

    lax.fori_loop(0, nchunk, body, 0)
    plsc.subcore_barrier()

    pltpu.sync_copy(agg_sp.at[pl.ds(s * rpt, rpt)],
                    agg_out.at[pl.ds(c * n_pad + s * rpt, rpt)])

  return k(p, src3, dst3)


def _edge_gather(z, src3, dst3):
  d = z.shape[1]
  nw, nchunk, cb = src3.shape
  epw = nchunk * cb
  e_pad = nw * epw
  mesh = plsc.VectorSubcoreMesh(core_axis_name="c", subcore_axis_name="s", num_cores=_NC, num_subcores=_NS)

  @functools.partial(
      pl.kernel,
      out_type=[
          jax.ShapeDtypeStruct((e_pad, d), jnp.float32),
          jax.ShapeDtypeStruct((e_pad, d), jnp.float32),
      ],
      mesh=mesh,
      compiler_params=pltpu.CompilerParams(use_tc_tiling_on_sc=False),
      scratch_types=[
          pltpu.VMEM((nchunk, cb), jnp.int32),
          pltpu.VMEM((nchunk, cb), jnp.int32),
          pltpu.VMEM((cb, d), jnp.float32),
          pltpu.VMEM((cb, d), jnp.float32),
          pltpu.SemaphoreType.DMA,
          pltpu.SemaphoreType.DMA,
      ],
  )
  def k(z_hbm, src_hbm, dst_hbm, ca_out, cb_out,
        sidx, didx, bufa, bufb, sema, semb):
    c = lax.axis_index("c")
    s = lax.axis_index("s")
    w = s * _NC + c

    pltpu.sync_copy(src_hbm.at[w], sidx)
    pltpu.sync_copy(dst_hbm.at[w], didx)

    def body(j, cacc):
      cpa = pltpu.async_copy(z_hbm.at[sidx.at[j]], bufa, sema)
      cpb = pltpu.async_copy(z_hbm.at[didx.at[j]], bufb, semb)
      cpa.wait()
      cpb.wait()
      base = w * epw + j * cb
      pltpu.sync_copy(bufa, ca_out.at[pl.ds(base, cb)])
      pltpu.sync_copy(bufb, cb_out.at[pl.ds(base, cb)])
      return cacc

    lax.fori_loop(0, nchunk, body, 0)

  return k(z, src3, dst3)


def kernel(x, edge_index, W1l, b1l, W1r, W2l, b2l, W2r, W3, b3, W4, b4):
  n, d_in = x.shape
  e = edge_index.shape[1]
  d_hid = W1l.shape[1]
  d_out = W2l.shape[1]

  e_pad = _cdiv(e, 16384) * 16384
  epw = e_pad // _NW
  n_pad = _cdiv(n + 1, _NS * _CB) * _NS * _CB

  src = edge_index[0]
  dst = edge_index[1]
  pad = e_pad - e
  src_p = jnp.concatenate([src, jnp.zeros((pad,), jnp.int32)])
  dst_p = jnp.concatenate([dst, jnp.full((pad,), n, jnp.int32)])
  src3 = src_p.reshape(_NW, epw // _CB, _CB)
  dst3 = dst_p.reshape(_NW, epw // _CB, _CB)

  p1, r1 = _mm2(x, W1l, W1r, rb=2000)
  agg1, deg = _seg_sum_deg(p1, src3, dst3, n_pad)
  a10, a11 = agg1[:n], agg1[n_pad:n_pad + n]
  d0, d1 = deg[:n], deg[n_pad:n_pad + n]

  p2, r2 = _combine_mm2(a10, a11, d0, d1, r1, b1l.reshape(1, d_hid),
                        W2l, W2r, rb=2000, relu=True)
  agg2 = _seg_sum(p2, src3, dst3, n_pad)
  a20, a21 = agg2[:n], agg2[n_pad:n_pad + n]
  z = _combine_z(a20, a21, d0, d1, r2, b2l.reshape(1, d_out), rb=2000)

  ca, cbuf = _edge_gather(z, src3, dst3)
  out = _decode(ca, cbuf, W3[:d_out], W3[d_out:],
                b3.reshape(1, 16), W4.reshape(1, 16), b4.reshape(1, 1),
                rb=16384)
  return out[:e]


# speedup vs baseline: 6.9291x; 1.1733x over previous
import functools

import jax
import jax.numpy as jnp
from jax import lax
from jax.experimental import pallas as pl
from jax.experimental.pallas import tpu as pltpu
from jax.experimental.pallas import tpu_sc as plsc

_NC = 2
_NS = 16
_NW = _NC * _NS
_CB = 128


def _cdiv(a, b):
  return (a + b - 1) // b


def _mm2(x, wl, wr, rb):
  m, k = x.shape
  dl, dr = wl.shape[1], wr.shape[1]

  def body(x_ref, wl_ref, wr_ref, ol_ref, or_ref):
    xv = x_ref[...]
    ol_ref[...] = jnp.dot(xv, wl_ref[...], preferred_element_type=jnp.float32)
    or_ref[...] = jnp.dot(xv, wr_ref[...], preferred_element_type=jnp.float32)

  return pl.pallas_call(
      body,
      grid=(m // rb,),
      in_specs=[
          pl.BlockSpec((rb, k), lambda i: (i, 0)),
          pl.BlockSpec((k, dl), lambda i: (0, 0)),
          pl.BlockSpec((k, dr), lambda i: (0, 0)),
      ],
      out_specs=[
          pl.BlockSpec((rb, dl), lambda i: (i, 0)),
          pl.BlockSpec((rb, dr), lambda i: (i, 0)),
      ],
      out_shape=[
          jax.ShapeDtypeStruct((m, dl), jnp.float32),
          jax.ShapeDtypeStruct((m, dr), jnp.float32),
      ],
  )(x, wl, wr)


def _combine_mm2(a0, a1, d0, d1, r, b, wl, wr, rb, relu):
  m, dh = r.shape
  dl, dr = wl.shape[1], wr.shape[1]

  def body(a0_ref, a1_ref, d0_ref, d1_ref, r_ref, b_ref, wl_ref, wr_ref,
           ol_ref, or_ref):
    deg = jnp.maximum((d0_ref[...] + d1_ref[...])[:, :1], 1.0)
    h = (a0_ref[...] + a1_ref[...]) / deg + r_ref[...] + b_ref[...]
    if relu:
      h = jnp.maximum(h, 0.0)
    ol_ref[...] = jnp.dot(h, wl_ref[...], preferred_element_type=jnp.float32)
    or_ref[...] = jnp.dot(h, wr_ref[...], preferred_element_type=jnp.float32)

  return pl.pallas_call(
      body,
      grid=(m // rb,),
      in_specs=[
          pl.BlockSpec((rb, dh), lambda i: (i, 0)),
          pl.BlockSpec((rb, dh), lambda i: (i, 0)),
          pl.BlockSpec((rb, 16), lambda i: (i, 0)),
          pl.BlockSpec((rb, 16), lambda i: (i, 0)),
          pl.BlockSpec((rb, dh), lambda i: (i, 0)),
          pl.BlockSpec((1, dh), lambda i: (0, 0)),
          pl.BlockSpec((dh, dl), lambda i: (0, 0)),
          pl.BlockSpec((dh, dr), lambda i: (0, 0)),
      ],
      out_specs=[
          pl.BlockSpec((rb, dl), lambda i: (i, 0)),
          pl.BlockSpec((rb, dr), lambda i: (i, 0)),
      ],
      out_shape=[
          jax.ShapeDtypeStruct((m, dl), jnp.float32),
          jax.ShapeDtypeStruct((m, dr), jnp.float32),
      ],
  )(a0, a1, d0, d1, r, b, wl, wr)


def _combine_z(a0, a1, d0, d1, r, b, rb):
  m, dh = r.shape

  def body(a0_ref, a1_ref, d0_ref, d1_ref, r_ref, b_ref, o_ref):
    deg = jnp.maximum((d0_ref[...] + d1_ref[...])[:, :1], 1.0)
    o_ref[...] = (a0_ref[...] + a1_ref[...]) / deg + r_ref[...] + b_ref[...]

  return pl.pallas_call(
      body,
      grid=(m // rb,),
      in_specs=[
          pl.BlockSpec((rb, dh), lambda i: (i, 0)),
          pl.BlockSpec((rb, dh), lambda i: (i, 0)),
          pl.BlockSpec((rb, 16), lambda i: (i, 0)),
          pl.BlockSpec((rb, 16), lambda i: (i, 0)),
          pl.BlockSpec((rb, dh), lambda i: (i, 0)),
          pl.BlockSpec((1, dh), lambda i: (0, 0)),
      ],
      out_specs=pl.BlockSpec((rb, dh), lambda i: (i, 0)),
      out_shape=jax.ShapeDtypeStruct((m, dh), jnp.float32),
  )(a0, a1, d0, d1, r, b)


def _decode(ca, cb, w3a, w3b, b3, w4, b4, rb):
  m = ca.shape[0]

  def body(ca_ref, cb_ref, w3a_ref, w3b_ref, b3_ref, w4_ref, b4_ref, o_ref):
    hid = jnp.maximum(
        jnp.dot(ca_ref[...], w3a_ref[...], preferred_element_type=jnp.float32)
        + jnp.dot(cb_ref[...], w3b_ref[...], preferred_element_type=jnp.float32)
        + b3_ref[...], 0.0)
    o_ref[...] = jnp.sum(hid * w4_ref[...], axis=1) + b4_ref[0, 0]

  return pl.pallas_call(
      body,
      grid=(m // rb,),
      in_specs=[
          pl.BlockSpec((rb, 16), lambda i: (i, 0)),
          pl.BlockSpec((rb, 16), lambda i: (i, 0)),
          pl.BlockSpec((16, 16), lambda i: (0, 0)),
          pl.BlockSpec((16, 16), lambda i: (0, 0)),
          pl.BlockSpec((1, 16), lambda i: (0, 0)),
          pl.BlockSpec((1, 16), lambda i: (0, 0)),
          pl.BlockSpec((1, 1), lambda i: (0, 0)),
      ],
      out_specs=pl.BlockSpec((rb,), lambda i: (i,)),
      out_shape=jax.ShapeDtypeStruct((m,), jnp.float32),
  )(ca, cb, w3a, w3b, b3, w4, b4)


def _zero_rows(buf, rows, d):
  z = jnp.zeros((16,), jnp.float32)

  def zf(i, c):
    for kk in range(d // 16):
      buf[i, pl.ds(kk * 16, 16)] = z
    return c

  lax.fori_loop(0, rows, zf, 0)


def _seg_sum(p, src3, dst3, n_pad, with_deg):
  d = p.shape[1]
  nw, nchunk, cb = src3.shape
  assert nchunk % 2 == 0 and nchunk >= 4
  rpt = n_pad // _NS
  mesh = plsc.VectorSubcoreMesh(core_axis_name="c", subcore_axis_name="s",
                                num_cores=_NC, num_subcores=_NS)

  out_type = [jax.ShapeDtypeStruct((_NC * n_pad, d), jnp.float32)]
  scratch = [
      pltpu.VMEM_SHARED((n_pad, d), jnp.float32),
      pltpu.VMEM((cb, d), jnp.float32),
      pltpu.VMEM((cb, d), jnp.float32),
      pltpu.VMEM((nchunk, cb), jnp.int32),
      pltpu.VMEM((nchunk, cb), jnp.int32),
      pltpu.SemaphoreType.DMA,
      pltpu.SemaphoreType.DMA,
  ]
  if with_deg:
    out_type.append(jax.ShapeDtypeStruct((_NC * n_pad, 16), jnp.float32))
    scratch += [
        pltpu.VMEM_SHARED((n_pad, 16), jnp.float32),
        pltpu.VMEM((cb, 16), jnp.float32),
        pltpu.SemaphoreType.DMA,
    ]

  @functools.partial(
      pl.kernel,
      out_type=out_type,
      mesh=mesh,
      compiler_params=pltpu.CompilerParams(use_tc_tiling_on_sc=False),
      scratch_types=scratch,
  )
  def k(p_hbm, src_hbm, dst_hbm, agg_out, *rest):
    if with_deg:
      (deg_out, agg_sp, g0, g1, sidx, didx, s0, s1, deg_sp, ones, dsem) = rest
    else:
      (agg_sp, g0, g1, sidx, didx, s0, s1) = rest
    c = lax.axis_index("c")
    s = lax.axis_index("s")
    w = s * _NC + c

    _zero_rows(g0, cb, d)
    for kk in range(rpt // cb):
      pltpu.sync_copy(g0, agg_sp.at[pl.ds(s * rpt + kk * cb, cb)])
    if with_deg:
      _zero_rows(ones, cb, 16)
      for kk in range(rpt // cb):
        pltpu.sync_copy(ones, deg_sp.at[pl.ds(s * rpt + kk * cb, cb)])
      one = jnp.ones((16,), jnp.float32)

      def of(i, cacc):
        ones[i, pl.ds(0, 16)] = one
        return cacc

      lax.fori_loop(0, cb, of, 0)

    pltpu.sync_copy(src_hbm.at[w], sidx)
    pltpu.sync_copy(dst_hbm.at[w], didx)
    plsc.subcore_barrier()

    bufs = (g0, g1)
    sems = (s0, s1)

    def start(j, b):
      pltpu.async_copy(p_hbm.at[sidx.at[j]], bufs[b], sems[b])

    def finish(j, b):
      pltpu.make_async_copy(p_hbm.at[sidx.at[j]], bufs[b], sems[b]).wait()
      pltpu.sync_copy(bufs[b], agg_sp.at[didx.at[j]], add=True)
      if with_deg:
        pltpu.async_copy(ones, deg_sp.at[didx.at[j]], dsem, add=True)

    start(0, 0)

    @pl.loop(0, nchunk - 2, step=2)
    def body(j):
      start(j + 1, 1)
      finish(j, 0)
      start(j + 2, 0)
      finish(j + 1, 1)

    start(nchunk - 1, 1)
    finish(nchunk - 2, 0)
    finish(nchunk - 1, 1)

    if with_deg:
      def drain(j, cacc):
        pltpu.make_async_copy(ones, deg_sp.at[didx.at[0]], dsem).wait()
        return cacc

      lax.fori_loop(0, nchunk, drain, 0)

    plsc.subcore_barrier()

    pltpu.sync_copy(agg_sp.at[pl.ds(s * rpt, rpt)],
                    agg_out.at[pl.ds(c * n_pad + s * rpt, rpt)])
    if with_deg:
      pltpu.sync_copy(deg_sp.at[pl.ds(s * rpt, rpt)],
                      deg_out.at[pl.ds(c * n_pad + s * rpt, rpt)])

  return k(p, src3, dst3)


def _edge_gather(z, src3, dst3):
  d = z.shape[1]
  nw, nchunk, cb = src3.shape
  assert nchunk % 2 == 0 and nchunk >= 4
  epw = nchunk * cb
  e_pad = nw * epw
  mesh = plsc.VectorSubcoreMesh(core_axis_name="c", subcore_axis_name="s",
                                num_cores=_NC, num_subcores=_NS)

  @functools.partial(
      pl.kernel,
      out_type=[
          jax.ShapeDtypeStruct((e_pad, d), jnp.float32),
          jax.ShapeDtypeStruct((e_pad, d), jnp.float32),
      ],
      mesh=mesh,
      compiler_params=pltpu.CompilerParams(use_tc_tiling_on_sc=False),
      scratch_types=[
          pltpu.VMEM((nchunk, cb), jnp.int32),
          pltpu.VMEM((nchunk, cb), jnp.int32),
          pltpu.VMEM((cb, d), jnp.float32),
          pltpu.VMEM((cb, d), jnp.float32),
          pltpu.VMEM((cb, d), jnp.float32),
          pltpu.VMEM((cb, d), jnp.float32),
          pltpu.SemaphoreType.DMA,
          pltpu.SemaphoreType.DMA,
          pltpu.SemaphoreType.DMA,
          pltpu.SemaphoreType.DMA,
      ],
  )
  def k(z_hbm, src_hbm, dst_hbm, ca_out, cb_out,
        sidx, didx, a0, a1, b0, b1, g0, g1, w0, w1):
    c = lax.axis_index("c")
    s = lax.axis_index("s")
    w = s * _NC + c

    pltpu.sync_copy(src_hbm.at[w], sidx)
    pltpu.sync_copy(dst_hbm.at[w], didx)

    abufs = (a0, a1)
    bbufs = (b0, b1)
    gsems = (g0, g1)
    wsems = (w0, w1)

    def start_gather(j, p):
      pltpu.async_copy(z_hbm.at[sidx.at[j]], abufs[p], gsems[p])
      pltpu.async_copy(z_hbm.at[didx.at[j]], bbufs[p], gsems[p])

    def wait_gather(j, p):
      pltpu.make_async_copy(z_hbm.at[sidx.at[j]], abufs[p], gsems[p]).wait()
      pltpu.make_async_copy(z_hbm.at[didx.at[j]], bbufs[p], gsems[p]).wait()

    def start_write(j, p):
      base = w * epw + j * cb
      pltpu.async_copy(abufs[p], ca_out.at[pl.ds(base, cb)], wsems[p])
      pltpu.async_copy(bbufs[p], cb_out.at[pl.ds(base, cb)], wsems[p])

    def wait_write(j, p):
      base = w * epw + j * cb
      pltpu.make_async_copy(abufs[p], ca_out.at[pl.ds(base, cb)],
                            wsems[p]).wait()
      pltpu.make_async_copy(bbufs[p], cb_out.at[pl.ds(base, cb)],
                            wsems[p]).wait()

    start_gather(0, 0)
    start_gather(1, 1)
    wait_gather(0, 0)
    start_write(0, 0)

    @pl.loop(1, nchunk - 1, step=2)
    def body(j):
      wait_write(j - 1, 0)
      start_gather(j + 1, 0)
      wait_gather(j, 1)
      start_write(j, 1)
      wait_write(j, 1)
      start_gather(j + 2, 1)
      wait_gather(j + 1, 0)
      start_write(j + 1, 0)

    wait_write(nchunk - 2, 0)
    wait_gather(nchunk - 1, 1)
    start_write(nchunk - 1, 1)
    wait_write(nchunk - 1, 1)

  return k(z, src3, dst3)


def kernel(x, edge_index, W1l, b1l, W1r, W2l, b2l, W2r, W3, b3, W4, b4):
  n, d_in = x.shape
  e = edge_index.shape[1]
  d_hid = W1l.shape[1]
  d_out = W2l.shape[1]

  e_pad = _cdiv(e, 16384) * 16384
  epw = e_pad // _NW
  n_pad = _cdiv(n + 1, _NS * _CB) * _NS * _CB

  src = edge_index[0]
  dst = edge_index[1]
  pad = e_pad - e
  src_p = jnp.concatenate([src, jnp.zeros((pad,), jnp.int32)])
  dst_p = jnp.concatenate([dst, jnp.full((pad,), n, jnp.int32)])
  src3 = src_p.reshape(_NW, epw // _CB, _CB)
  dst3 = dst_p.reshape(_NW, epw // _CB, _CB)

  p1, r1 = _mm2(x, W1l, W1r, rb=2000)
  agg1, deg = _seg_sum(p1, src3, dst3, n_pad, with_deg=True)
  a10, a11 = agg1[:n], agg1[n_pad:n_pad + n]
  d0, d1 = deg[:n], deg[n_pad:n_pad + n]

  p2, r2 = _combine_mm2(a10, a11, d0, d1, r1, b1l.reshape(1, d_hid),
                        W2l, W2r, rb=2000, relu=True)
  agg2, = _seg_sum(p2, src3, dst3, n_pad, with_deg=False)
  a20, a21 = agg2[:n], agg2[n_pad:n_pad + n]
  z = _combine_z(a20, a21, d0, d1, r2, b2l.reshape(1, d_out), rb=2000)

  ca, cbuf = _edge_gather(z, src3, dst3)
  out = _decode(ca, cbuf, W3[:d_out], W3[d_out:],
                b3.reshape(1, 16), W4.reshape(1, 16), b4.reshape(1, 1),
                rb=16384)
  return out[:e]


# speedup vs baseline: 7.1812x; 1.0364x over previous
import functools

import jax
import jax.numpy as jnp
from jax import lax
from jax.experimental import pallas as pl
from jax.experimental.pallas import tpu as pltpu
from jax.experimental.pallas import tpu_sc as plsc

_NC = 2
_NS = 16
_NW = _NC * _NS
_CB = 128


def _cdiv(a, b):
  return (a + b - 1) // b


def _mm2(x, wl, wr, b, rb):
  m, k = x.shape
  dl, dr = wl.shape[1], wr.shape[1]

  def body(x_ref, wl_ref, wr_ref, b_ref, ol_ref, or_ref):
    xv = x_ref[...]
    ol_ref[...] = jnp.dot(xv, wl_ref[...], preferred_element_type=jnp.float32)
    or_ref[...] = jnp.dot(xv, wr_ref[...],
                          preferred_element_type=jnp.float32) + b_ref[...]

  return pl.pallas_call(
      body,
      grid=(m // rb,),
      in_specs=[
          pl.BlockSpec((rb, k), lambda i: (i, 0)),
          pl.BlockSpec((k, dl), lambda i: (0, 0)),
          pl.BlockSpec((k, dr), lambda i: (0, 0)),
          pl.BlockSpec((1, dr), lambda i: (0, 0)),
      ],
      out_specs=[
          pl.BlockSpec((rb, dl), lambda i: (i, 0)),
          pl.BlockSpec((rb, dr), lambda i: (i, 0)),
      ],
      out_shape=[
          jax.ShapeDtypeStruct((m, dl), jnp.float32),
          jax.ShapeDtypeStruct((m, dr), jnp.float32),
      ],
  )(x, wl, wr, b)


def _combine_proj(a0, a1, d0, d1, r1b, w2l, w2r, w3, b2l, b3, rb):
  m, dh = r1b.shape

  def body(a0_ref, a1_ref, d0_ref, d1_ref, r_ref, w2l_ref, w2r_ref, w3_ref,
           b2l_ref, b3_ref, p_ref, rc_ref):
    deg = jnp.maximum((d0_ref[...] + d1_ref[...])[:, :1], 1.0)
    h = jnp.maximum((a0_ref[...] + a1_ref[...]) / deg + r_ref[...], 0.0)
    w3a = w3_ref[0:16, :]
    w3b = w3_ref[16:32, :]
    dot = lambda p, q: jnp.dot(p, q, preferred_element_type=jnp.float32)
    p_ref[:, 0:16] = dot(h, dot(w2l_ref[...], w3a))
    p_ref[:, 16:32] = dot(h, dot(w2l_ref[...], w3b))
    rc_ref[:, 0:16] = (dot(h, dot(w2r_ref[...], w3a))
                       + dot(b2l_ref[...], w3a) + b3_ref[...])
    rc_ref[:, 16:32] = dot(h, dot(w2r_ref[...], w3b)) + dot(b2l_ref[...], w3b)

  return pl.pallas_call(
      body,
      grid=(m // rb,),
      in_specs=[
          pl.BlockSpec((rb, dh), lambda i: (i, 0)),
          pl.BlockSpec((rb, dh), lambda i: (i, 0)),
          pl.BlockSpec((rb, 16), lambda i: (i, 0)),
          pl.BlockSpec((rb, 16), lambda i: (i, 0)),
          pl.BlockSpec((rb, dh), lambda i: (i, 0)),
          pl.BlockSpec((dh, 16), lambda i: (0, 0)),
          pl.BlockSpec((dh, 16), lambda i: (0, 0)),
          pl.BlockSpec((32, 16), lambda i: (0, 0)),
          pl.BlockSpec((1, 16), lambda i: (0, 0)),
          pl.BlockSpec((1, 16), lambda i: (0, 0)),
      ],
      out_specs=[
          pl.BlockSpec((rb, 32), lambda i: (i, 0)),
          pl.BlockSpec((rb, 32), lambda i: (i, 0)),
      ],
      out_shape=[
          jax.ShapeDtypeStruct((m, 32), jnp.float32),
          jax.ShapeDtypeStruct((m, 32), jnp.float32),
      ],
  )(a0, a1, d0, d1, r1b, w2l, w2r, w3, b2l, b3)


def _decode(cu, cv, w4, b4, rb):
  m = cu.shape[0]

  def body(cu_ref, cv_ref, w4_ref, b4_ref, o_ref):
    hid = jnp.maximum(cu_ref[...] + cv_ref[...], 0.0)
    o_ref[...] = jnp.sum(hid * w4_ref[...], axis=1) + b4_ref[0, 0]

  return pl.pallas_call(
      body,
      grid=(m // rb,),
      in_specs=[
          pl.BlockSpec((rb, 16), lambda i: (i, 0)),
          pl.BlockSpec((rb, 16), lambda i: (i, 0)),
          pl.BlockSpec((1, 16), lambda i: (0, 0)),
          pl.BlockSpec((1, 1), lambda i: (0, 0)),
      ],
      out_specs=pl.BlockSpec((rb,), lambda i: (i,)),
      out_shape=jax.ShapeDtypeStruct((m,), jnp.float32),
  )(cu, cv, w4, b4)


def _zero_rows(buf, rows, d):
  z = jnp.zeros((16,), jnp.float32)

  def zf(i, c):
    for kk in range(d // 16):
      buf[i, pl.ds(kk * 16, 16)] = z
    return c

  lax.fori_loop(0, rows, zf, 0)


def _seg_sum(p, src3, dst3, n_pad, with_deg):
  d = p.shape[1]
  nw, nchunk, cb = src3.shape
  assert nchunk % 2 == 0 and nchunk >= 4
  rpt = n_pad // _NS
  mesh = plsc.VectorSubcoreMesh(core_axis_name="c", subcore_axis_name="s",
                                num_cores=_NC, num_subcores=_NS)

  out_type = [jax.ShapeDtypeStruct((_NC * n_pad, d), jnp.float32)]
  scratch = [
      pltpu.VMEM_SHARED((n_pad, d), jnp.float32),
      pltpu.VMEM((cb, d), jnp.float32),
      pltpu.VMEM((cb, d), jnp.float32),
      pltpu.VMEM((nchunk, cb), jnp.int32),
      pltpu.VMEM((nchunk, cb), jnp.int32),
      pltpu.SemaphoreType.DMA,
      pltpu.SemaphoreType.DMA,
  ]
  if with_deg:
    out_type.append(jax.ShapeDtypeStruct((_NC * n_pad, 16), jnp.float32))
    scratch += [
        pltpu.VMEM_SHARED((n_pad, 16), jnp.float32),
        pltpu.VMEM((cb, 16), jnp.float32),
        pltpu.SemaphoreType.DMA,
    ]

  @functools.partial(
      pl.kernel,
      out_type=out_type,
      mesh=mesh,
      compiler_params=pltpu.CompilerParams(use_tc_tiling_on_sc=False),
      scratch_types=scratch,
  )
  def k(p_hbm, src_hbm, dst_hbm, agg_out, *rest):
    if with_deg:
      (deg_out, agg_sp, g0, g1, sidx, didx, s0, s1, deg_sp, ones, dsem) = rest
    else:
      (agg_sp, g0, g1, sidx, didx, s0, s1) = rest
    c = lax.axis_index("c")
    s = lax.axis_index("s")
    w = s * _NC + c

    _zero_rows(g0, cb, d)
    for kk in range(rpt // cb):
      pltpu.sync_copy(g0, agg_sp.at[pl.ds(s * rpt + kk * cb, cb)])
    if with_deg:
      _zero_rows(ones, cb, 16)
      for kk in range(rpt // cb):
        pltpu.sync_copy(ones, deg_sp.at[pl.ds(s * rpt + kk * cb, cb)])
      one = jnp.ones((16,), jnp.float32)

      def of(i, cacc):
        ones[i, pl.ds(0, 16)] = one
        return cacc

      lax.fori_loop(0, cb, of, 0)

    pltpu.sync_copy(src_hbm.at[w], sidx)
    pltpu.sync_copy(dst_hbm.at[w], didx)
    plsc.subcore_barrier()

    bufs = (g0, g1)
    sems = (s0, s1)

    def start(j, b):
      pltpu.async_copy(p_hbm.at[sidx.at[j]], bufs[b], sems[b])

    def finish(j, b):
      pltpu.make_async_copy(p_hbm.at[sidx.at[j]], bufs[b], sems[b]).wait()
      pltpu.sync_copy(bufs[b], agg_sp.at[didx.at[j]], add=True)
      if with_deg:
        pltpu.async_copy(ones, deg_sp.at[didx.at[j]], dsem, add=True)

    start(0, 0)

    @pl.loop(0, nchunk - 2, step=2)
    def body(j):
      start(j + 1, 1)
      finish(j, 0)
      start(j + 2, 0)
      finish(j + 1, 1)

    start(nchunk - 1, 1)
    finish(nchunk - 2, 0)
    finish(nchunk - 1, 1)

    if with_deg:
      def drain(j, cacc):
        pltpu.make_async_copy(ones, deg_sp.at[didx.at[0]], dsem).wait()
        return cacc

      lax.fori_loop(0, nchunk, drain, 0)

    plsc.subcore_barrier()

    pltpu.sync_copy(agg_sp.at[pl.ds(s * rpt, rpt)],
                    agg_out.at[pl.ds(c * n_pad + s * rpt, rpt)])
    if with_deg:
      pltpu.sync_copy(deg_sp.at[pl.ds(s * rpt, rpt)],
                      deg_out.at[pl.ds(c * n_pad + s * rpt, rpt)])

  return k(p, src3, dst3)


def _uv_edge_gather(aggcat, deg, rcat, src3, dst3, n_pad):
  nw, nchunk, cb = src3.shape
  assert nchunk % 2 == 0 and nchunk >= 4
  epw = nchunk * cb
  e_pad = nw * epw
  rpt = n_pad // _NS
  nblk = rpt // cb
  mesh = plsc.VectorSubcoreMesh(core_axis_name="c", subcore_axis_name="s",
                                num_cores=_NC, num_subcores=_NS)

  @functools.partial(
      pl.kernel,
      out_type=[
          jax.ShapeDtypeStruct((e_pad, 16), jnp.float32),
          jax.ShapeDtypeStruct((e_pad, 16), jnp.float32),
      ],
      mesh=mesh,
      compiler_params=pltpu.CompilerParams(use_tc_tiling_on_sc=False),
      scratch_types=[
          pltpu.VMEM_SHARED((n_pad, 16), jnp.float32),
          pltpu.VMEM_SHARED((n_pad, 16), jnp.float32),
          pltpu.VMEM((nchunk, cb), jnp.int32),
          pltpu.VMEM((nchunk, cb), jnp.int32),
          pltpu.VMEM((cb, 32), jnp.float32),
          pltpu.VMEM((cb, 32), jnp.float32),
          pltpu.VMEM((cb, 32), jnp.float32),
          pltpu.VMEM((cb, 16), jnp.float32),
          pltpu.VMEM((cb, 16), jnp.float32),
          pltpu.VMEM((cb, 16), jnp.float32),
          pltpu.VMEM((cb, 16), jnp.float32),
          pltpu.VMEM((cb, 16), jnp.float32),
          pltpu.VMEM((cb, 16), jnp.float32),
          pltpu.SemaphoreType.DMA,
          pltpu.SemaphoreType.DMA,
          pltpu.SemaphoreType.DMA,
          pltpu.SemaphoreType.DMA,
      ],
  )
  def k(agg_hbm, deg_hbm, rcat_hbm, src_hbm, dst_hbm, cu_out, cv_out,
        utab, vtab, sidx, didx, pa0, pa1, prc, pd0, pd1,
        u0, u1, v0, v1, g0, g1, w0, w1):
    c = lax.axis_index("c")
    s = lax.axis_index("s")
    w = s * _NC + c

    for blk in range(nblk):
      row0 = s * rpt + blk * cb
      pltpu.sync_copy(agg_hbm.at[pl.ds(row0, cb)], pa0)
      pltpu.sync_copy(agg_hbm.at[pl.ds(n_pad + row0, cb)], pa1)
      pltpu.sync_copy(deg_hbm.at[pl.ds(row0, cb)], pd0)
      pltpu.sync_copy(deg_hbm.at[pl.ds(n_pad + row0, cb)], pd1)
      pltpu.sync_copy(rcat_hbm.at[pl.ds(row0, cb)], prc)

      def rowf(i, cacc):
        rec = 1.0 / jnp.maximum(pd0[i, pl.ds(0, 16)] + pd1[i, pl.ds(0, 16)],
                                1.0)
        u0[i, pl.ds(0, 16)] = (
            (pa0[i, pl.ds(0, 16)] + pa1[i, pl.ds(0, 16)]) * rec
            + prc[i, pl.ds(0, 16)])
        v0[i, pl.ds(0, 16)] = (
            (pa0[i, pl.ds(16, 16)] + pa1[i, pl.ds(16, 16)]) * rec
            + prc[i, pl.ds(16, 16)])
        return cacc

      lax.fori_loop(0, cb, rowf, 0)
      pltpu.sync_copy(u0, utab.at[pl.ds(row0, cb)])
      pltpu.sync_copy(v0, vtab.at[pl.ds(row0, cb)])

    pltpu.sync_copy(src_hbm.at[w], sidx)
    pltpu.sync_copy(dst_hbm.at[w], didx)
    plsc.subcore_barrier()

    ubufs = (u0, u1)
    vbufs = (v0, v1)
    gsems = (g0, g1)
    wsems = (w0, w1)

    def start_gather(j, p):
      pltpu.async_copy(utab.at[sidx.at[j]], ubufs[p], gsems[p])
      pltpu.async_copy(vtab.at[didx.at[j]], vbufs[p], gsems[p])

    def wait_gather(j, p):
      pltpu.make_async_copy(utab.at[sidx.at[j]], ubufs[p], gsems[p]).wait()
      pltpu.make_async_copy(vtab.at[didx.at[j]], vbufs[p], gsems[p]).wait()

    def start_write(j, p):
      base = w * epw + j * cb
      pltpu.async_copy(ubufs[p], cu_out.at[pl.ds(base, cb)], wsems[p])
      pltpu.async_copy(vbufs[p], cv_out.at[pl.ds(base, cb)], wsems[p])

    def wait_write(j, p):
      base = w * epw + j * cb
      pltpu.make_async_copy(ubufs[p], cu_out.at[pl.ds(base, cb)],
                            wsems[p]).wait()
      pltpu.make_async_copy(vbufs[p], cv_out.at[pl.ds(base, cb)],
                            wsems[p]).wait()

    start_gather(0, 0)
    start_gather(1, 1)
    wait_gather(0, 0)
    start_write(0, 0)

    @pl.loop(1, nchunk - 1, step=2)
    def body(j):
      wait_write(j - 1, 0)
      start_gather(j + 1, 0)
      wait_gather(j, 1)
      start_write(j, 1)
      wait_write(j, 1)
      start_gather(j + 2, 1)
      wait_gather(j + 1, 0)
      start_write(j + 1, 0)

    wait_write(nchunk - 2, 0)
    wait_gather(nchunk - 1, 1)
    start_write(nchunk - 1, 1)
    wait_write(nchunk - 1, 1)

  return k(aggcat, deg, rcat, src3, dst3)


def kernel(x, edge_index, W1l, b1l, W1r, W2l, b2l, W2r, W3, b3, W4, b4):
  n, d_in = x.shape
  e = edge_index.shape[1]
  d_hid = W1l.shape[1]

  e_pad = _cdiv(e, 16384) * 16384
  epw = e_pad // _NW
  n_pad = _cdiv(n + 1, _NS * _CB) * _NS * _CB

  src = edge_index[0]
  dst = edge_index[1]
  pad = e_pad - e
  src_p = jnp.concatenate([src, jnp.zeros((pad,), jnp.int32)])
  dst_p = jnp.concatenate([dst, jnp.full((pad,), n, jnp.int32)])
  src3 = src_p.reshape(_NW, epw // _CB, _CB)
  dst3 = dst_p.reshape(_NW, epw // _CB, _CB)

  p1, r1b = _mm2(x, W1l, W1r, b1l.reshape(1, d_hid), rb=2000)
  agg1, deg = _seg_sum(p1, src3, dst3, n_pad, with_deg=True)
  a10, a11 = agg1[:n], agg1[n_pad:n_pad + n]
  d0, d1 = deg[:n], deg[n_pad:n_pad + n]

  pcat, rcat = _combine_proj(a10, a11, d0, d1, r1b, W2l, W2r, W3,
                             b2l.reshape(1, 16), b3.reshape(1, 16), rb=2000)
  aggcat, = _seg_sum(pcat, src3, dst3, n_pad, with_deg=False)

  rcat_p = jnp.concatenate(
      [rcat, jnp.zeros((n_pad - n, 32), jnp.float32)], axis=0)
  cu, cv = _uv_edge_gather(aggcat, deg, rcat_p, src3, dst3, n_pad)
  out = _decode(cu, cv, W4.reshape(1, 16), b4.reshape(1, 1), rb=16384)
  return out[:e]
